# baseline (speedup 1.0000x reference)
import functools

import jax
import jax.numpy as jnp
from jax import lax
from jax.experimental import pallas as pl
from jax.experimental.pallas import tpu as pltpu
from jax.experimental.pallas import tpu_sc as plsc

F32 = jnp.float32
BF16 = jnp.bfloat16


def _fit(b, n):
    b = min(b, n)
    while n % b:
        b -= 1
    return b


def _mm_body(x_ref, w_ref, ss_ref, b_ref, res_ref, o_ref, *, ln, act, D):
    x = x_ref[...].astype(F32)
    if act == "silu":
        x = x * jax.nn.sigmoid(x)
    if ln:
        mu = jnp.mean(x, axis=1, keepdims=True)
        var = jnp.mean((x - mu) * (x - mu), axis=1, keepdims=True)
        xh = (x - mu) * jax.lax.rsqrt(var + 1e-5)
        ss = ss_ref[0]
        x = xh * (1.0 + ss[:, D:]) + ss[:, :D]
    w = w_ref[...]
    if w.dtype == BF16:
        x = x.astype(BF16)
    out = jnp.dot(x, w, preferred_element_type=F32)
    if b_ref is not None:
        out = out + b_ref[0]
    if res_ref is not None:
        out = out + res_ref[...]
    o_ref[...] = out


def _mm(x, w, ss=None, bias=None, res=None, *, ln=False, act=None,
        bm=512, bn=512, rows_per_batch=None, interpret=False):
    M, K = x.shape
    N = w.shape[1]
    bm = _fit(bm, M)
    if rows_per_batch is not None:
        bm = _fit(bm, rows_per_batch)
    bn = _fit(bn, N)
    grid = (M // bm, N // bn)
    in_specs = [
        pl.BlockSpec((bm, K), lambda m, n: (m, 0)),
        pl.BlockSpec((K, bn), lambda m, n: (0, n)),
    ]
    args = [x, w]
    if ss is not None:
        rpb = rows_per_batch // bm
        in_specs.append(pl.BlockSpec((1, 1, ss.shape[-1]),
                                     lambda m, n: (m // rpb, 0, 0)))
        args.append(ss)
    if bias is not None:
        in_specs.append(pl.BlockSpec((1, 1, bn), lambda m, n: (0, 0, n)))
        args.append(bias.reshape(1, 1, N))
    if res is not None:
        in_specs.append(pl.BlockSpec((bm, bn), lambda m, n: (m, n)))
        args.append(res)

    def body(*refs):
        x_ref, w_ref = refs[0], refs[1]
        i = 2
        ss_ref = b_ref = res_ref = None
        if ss is not None:
            ss_ref = refs[i]; i += 1
        if bias is not None:
            b_ref = refs[i]; i += 1
        if res is not None:
            res_ref = refs[i]; i += 1
        _mm_body(x_ref, w_ref, ss_ref, b_ref, res_ref, refs[-1],
                 ln=ln, act=act, D=K)

    return pl.pallas_call(
        body,
        grid=grid,
        in_specs=in_specs,
        out_specs=pl.BlockSpec((bm, bn), lambda m, n: (m, n)),
        out_shape=jax.ShapeDtypeStruct((M, N), F32),
        compiler_params=pltpu.CompilerParams(
            dimension_semantics=("parallel", "parallel")),
        interpret=interpret,
    )(*args)


def _attn_body(q_ref, k_ref, v_ref, o_ref, *, scale, dh):
    qq = (q_ref[...].astype(F32) * scale).astype(BF16)
    kk = k_ref[...].astype(BF16)
    vv = v_ref[...].astype(BF16)
    outs = []
    for i in (0, 1):
        q = qq[:, i * dh:(i + 1) * dh]
        k = kk[:, i * dh:(i + 1) * dh]
        v = vv[:, i * dh:(i + 1) * dh]
        s = jax.lax.dot_general(q, k, (((1,), (1,)), ((), ())),
                                preferred_element_type=F32)
        m = jnp.max(s, axis=1, keepdims=True)
        p = jnp.exp(s - m)
        l = jnp.sum(p, axis=1, keepdims=True)
        p = (p / l).astype(BF16)
        outs.append(jnp.dot(p, v, preferred_element_type=F32))
    o_ref[...] = jnp.concatenate(outs, axis=1)


def _attn(q_arr, kv_arr, *, B, H, dh, sq, sk, qc0, kc0, vc0, bq=512,
          interpret=False):
    bq = _fit(bq, sq)
    nq = sq // bq
    HP = H // 2
    grid = (B * HP, nq)
    return pl.pallas_call(
        functools.partial(_attn_body, scale=1.0 / (dh ** 0.5), dh=dh),
        grid=grid,
        in_specs=[
            pl.BlockSpec((bq, 2 * dh),
                         lambda bh, i: ((bh // HP) * nq + i, qc0 + bh % HP)),
            pl.BlockSpec((sk, 2 * dh),
                         lambda bh, i: (bh // HP, kc0 + bh % HP)),
            pl.BlockSpec((sk, 2 * dh),
                         lambda bh, i: (bh // HP, vc0 + bh % HP)),
        ],
        out_specs=pl.BlockSpec((bq, 2 * dh),
                               lambda bh, i: ((bh // HP) * nq + i, bh % HP)),
        out_shape=jax.ShapeDtypeStruct((B * sq, H * dh), F32),
        compiler_params=pltpu.CompilerParams(
            dimension_semantics=("parallel", "parallel")),
        interpret=interpret,
    )(q_arr, kv_arr, kv_arr)


def _pre_moe_body(h2_ref, ss_ref, gw_ref, lt_ref, xn_ref, coef_ref, *, D, E):
    h2 = h2_ref[...]
    mu = jnp.mean(h2, axis=1, keepdims=True)
    var = jnp.mean((h2 - mu) * (h2 - mu), axis=1, keepdims=True)
    xh = (h2 - mu) * jax.lax.rsqrt(var + 1e-5)
    ss = ss_ref[0]
    xn_ref[...] = xh * (1.0 + ss[:, D:]) + ss[:, :D]

    logits = jnp.dot(h2, gw_ref[...], preferred_element_type=F32) + lt_ref[0]
    col = jax.lax.broadcasted_iota(jnp.int32, logits.shape, 1)
    valid = col < E
    neg = jnp.float32(-1e30)
    lg = jnp.where(valid, logits, neg)
    mx = jnp.max(lg, axis=1, keepdims=True)
    ex = jnp.where(valid, jnp.exp(lg - mx), 0.0)
    scores = ex / jnp.sum(ex, axis=1, keepdims=True)
    gate_on = (jax.nn.sigmoid(logits) - 0.5 > 0.0) & valid
    coef_ref[...] = jnp.where(gate_on, scores, 0.0)


def _pre_moe(h2, moe_ss, gw_pad, lt_pad, *, rows_per_batch, E, bm=512,
             interpret=False):
    S, D = h2.shape
    P = gw_pad.shape[1]
    bm = _fit(bm, rows_per_batch)
    rpb = rows_per_batch // bm
    return pl.pallas_call(
        functools.partial(_pre_moe_body, D=D, E=E),
        grid=(S // bm,),
        in_specs=[
            pl.BlockSpec((bm, D), lambda m: (m, 0)),
            pl.BlockSpec((1, 1, 2 * D), lambda m: (m // rpb, 0, 0)),
            pl.BlockSpec((D, P), lambda m: (0, 0)),
            pl.BlockSpec((1, 1, P), lambda m: (m // rpb, 0, 0)),
        ],
        out_specs=[
            pl.BlockSpec((bm, D), lambda m: (m, 0)),
            pl.BlockSpec((bm, P), lambda m: (m, 0)),
        ],
        out_shape=[
            jax.ShapeDtypeStruct((S, D), F32),
            jax.ShapeDtypeStruct((S, P), F32),
        ],
        compiler_params=pltpu.CompilerParams(
            dimension_semantics=("parallel",)),
        interpret=interpret,
    )(h2, moe_ss, gw_pad, lt_pad)


def _gmoe_body(nb_ref, off_ref, xg_ref, w1_ref, b1_ref, w2_ref, b2_ref,
               cg_ref, o_ref, acc_ref, obuf_ref, sem, *, C, bm, PMAX):
    e = pl.program_id(0)
    c = pl.program_id(1)
    mi = pl.program_id(2)

    @pl.when((e == 0) & (c == 0) & (mi == 0))
    def _zero_pad_row_block():
        obuf_ref[...] = jnp.zeros_like(obuf_ref)
        cp = pltpu.make_async_copy(obuf_ref, o_ref.at[pl.ds(PMAX, bm), :],
                                   sem)
        cp.start()
        cp.wait()

    @pl.when(mi < nb_ref[e])
    def _active():
        h = jnp.dot(xg_ref[...].astype(BF16), w1_ref[0].astype(BF16),
                    preferred_element_type=F32) + b1_ref[0]
        h = jax.nn.gelu(h, approximate=True)
        eo = jnp.dot(h.astype(BF16), w2_ref[0].astype(BF16),
                     preferred_element_type=F32)
        base = mi * bm

        @pl.when(c == 0)
        def _init():
            acc_ref[pl.ds(base, bm), :] = eo

        @pl.when(c > 0)
        def _accum():
            acc_ref[pl.ds(base, bm), :] += eo

        @pl.when(c == C - 1)
        def _flush():
            cf = cg_ref[:, 0:1]
            full = acc_ref[pl.ds(base, bm), :] + b2_ref[0]
            obuf_ref[...] = cf * full
            j = off_ref[e] + mi
            cp = pltpu.make_async_copy(
                obuf_ref, o_ref.at[pl.ds(j * bm, bm), :], sem)
            cp.start()
            cp.wait()


def _gmoe(xg, w1, b1, w2, b2, coef_g8, nb, off_blk, *, bm, bc=1024,
          interpret=False):
    PMAX, D = xg.shape
    E, _, ff = w1.shape
    bc = _fit(bc, ff)
    C = ff // bc
    JBMAX = PMAX // bm
    S = PMAX // E
    MImax = S // bm

    def _jeff(e, nb, off):
        last = jnp.maximum(nb[e] - 1, 0)
        return lambda mi: jnp.minimum(off[e] + jnp.minimum(mi, last),
                                      JBMAX - 1)

    def xg_map(e, c, mi, nb, off):
        return (_jeff(e, nb, off)(mi), 0)

    def cg_map(e, c, mi, nb, off):
        return (_jeff(e, nb, off)(mi), 0)

    grid_spec = pltpu.PrefetchScalarGridSpec(
        num_scalar_prefetch=2,
        grid=(E, C, MImax),
        in_specs=[
            pl.BlockSpec((bm, D), xg_map),
            pl.BlockSpec((1, D, bc), lambda e, c, mi, nb, off: (e, 0, c)),
            pl.BlockSpec((1, 1, bc), lambda e, c, mi, nb, off: (e, 0, c)),
            pl.BlockSpec((1, bc, D), lambda e, c, mi, nb, off: (e, c, 0)),
            pl.BlockSpec((1, 1, D), lambda e, c, mi, nb, off: (e, 0, 0)),
            pl.BlockSpec((bm, 8), cg_map),
        ],
        out_specs=pl.BlockSpec(memory_space=pltpu.MemorySpace.HBM),
        scratch_shapes=[
            pltpu.VMEM((S, D), F32),
            pltpu.VMEM((bm, D), F32),
            pltpu.SemaphoreType.DMA,
        ],
    )
    return pl.pallas_call(
        functools.partial(_gmoe_body, C=C, bm=bm, PMAX=PMAX),
        grid_spec=grid_spec,
        out_shape=jax.ShapeDtypeStruct((PMAX + bm, D), F32),
        compiler_params=pltpu.CompilerParams(
            dimension_semantics=("arbitrary", "arbitrary", "arbitrary")),
        interpret=interpret,
    )(nb, off_blk, xg, w1, b1.reshape(E, 1, ff), w2, b2.reshape(E, 1, D),
      coef_g8)


def _sc_gather(table, idx, out_rows, *, interpret=False):
    if interpret:
        return jnp.take(table, idx, axis=0)[:out_rows]

    sl = table.shape[1]
    dt = table.dtype
    K = 64
    info = plsc.get_sparse_core_info()
    NC, NS = info.num_cores, info.num_subcores
    NW = NC * NS
    mesh = plsc.VectorSubcoreMesh(core_axis_name="c", subcore_axis_name="s")

    @functools.partial(
        pl.kernel, mesh=mesh,
        out_type=jax.ShapeDtypeStruct((out_rows, sl, 128), dt),
        scratch_types=[
            pltpu.VMEM((K,), jnp.int32),
            pltpu.VMEM((K, sl, 128), dt),
            pltpu.SemaphoreType.DMA,
        ],
    )
    def gather_kernel(table_hbm, idx_hbm, out_hbm, idx_v, rows_v, sem):
        wid = lax.axis_index("s") * NC + lax.axis_index("c")
        total_chunks = out_rows // K
        my_chunks = (total_chunks - wid + NW - 1) // NW

        def body(i, _):
            base = (wid + i * NW) * K
            pltpu.sync_copy(idx_hbm.at[pl.ds(base, K)], idx_v)
            pltpu.async_copy(table_hbm.at[idx_v], rows_v, sem).wait()
            pltpu.sync_copy(rows_v, out_hbm.at[pl.ds(base, K)])
            return 0

        lax.fori_loop(0, my_chunks, body, 0)

    return gather_kernel(table, idx)


def _combine_body(*refs):
    h2_ref, coef_ref = refs[0], refs[1]
    ge_refs = refs[2:-1]
    o_ref = refs[-1]
    acc = h2_ref[...]
    cb = coef_ref[...]
    for e, g_ref in enumerate(ge_refs):
        on = (cb[:, e:e + 1] > 0.0).astype(F32)
        acc = acc + on * g_ref[...].astype(F32)
    o_ref[...] = acc


def _combine(h2, coef, geall, E, *, bm=256, interpret=False):
    S, D = h2.shape
    P = coef.shape[1]
    bm = _fit(bm, S)
    MB = S // bm
    spec = pl.BlockSpec((bm, D), lambda m: (m, 0))
    ge_specs = [pl.BlockSpec((bm, D), lambda m, e=e: (e * MB + m, 0))
                for e in range(E)]
    return pl.pallas_call(
        _combine_body,
        grid=(MB,),
        in_specs=[spec, pl.BlockSpec((bm, P), lambda m: (m, 0))] + ge_specs,
        out_specs=spec,
        out_shape=jax.ShapeDtypeStruct((S, D), F32),
        compiler_params=pltpu.CompilerParams(
            dimension_semantics=("parallel",)),
        interpret=interpret,
    )(h2, coef, *([geall] * E))


def _add_body(a_ref, b_ref, o_ref):
    o_ref[...] = a_ref[...] + b_ref[...]


def _add(a, b, *, bm=1024, interpret=False):
    M, N = a.shape
    bm = _fit(bm, M)
    return pl.pallas_call(
        _add_body,
        grid=(M // bm,),
        in_specs=[pl.BlockSpec((bm, N), lambda m: (m, 0)),
                  pl.BlockSpec((bm, N), lambda m: (m, 0))],
        out_specs=pl.BlockSpec((bm, N), lambda m: (m, 0)),
        out_shape=jax.ShapeDtypeStruct((M, N), F32),
        compiler_params=pltpu.CompilerParams(
            dimension_semantics=("parallel",)),
        interpret=interpret,
    )(a, b)


def _forward(x, y, emb, sa_Wqkv, sa_Wo, sa_modW, sa_modb, ca_Wq, ca_Wk,
             ca_Wv, ca_Wo, ca_modW, ca_modb, gate_w, text_gate_w, moe_modW,
             moe_modb, exp_W1, exp_b1, exp_W2, exp_b2, interpret=False):
    H = 16
    E = gate_w.shape[0]
    B, s, D = x.shape
    dh = D // H
    sy = y.shape[1]
    S = B * s
    P = 128

    modW = jnp.concatenate([sa_modW, ca_modW, moe_modW], axis=1)
    modb = jnp.concatenate([sa_modb, ca_modb, moe_modb])
    ss_all = _mm(emb, modW, bias=modb, act="silu", bm=B, bn=1024,
                 interpret=interpret)
    sa_ss = ss_all[:, 0 * D:2 * D].reshape(B, 1, 2 * D)
    ca_ss = ss_all[:, 2 * D:4 * D].reshape(B, 1, 2 * D)
    moe_ss = ss_all[:, 4 * D:6 * D].reshape(B, 1, 2 * D)

    gw_pad = jnp.pad(text_gate_w.T, ((0, 0), (0, P - E)))
    lt_pad = _mm(emb, gw_pad, bm=B, bn=P, interpret=interpret).reshape(B, 1, P)

    x2 = x.reshape(S, D)
    qkv = _mm(x2, sa_Wqkv.astype(BF16), ss=sa_ss, ln=True,
              rows_per_batch=s, interpret=interpret)
    o = _attn(qkv, qkv, B=B, H=H, dh=dh, sq=s, sk=s,
              qc0=0, kc0=H // 2, vc0=H, interpret=interpret)
    h1 = _mm(o, sa_Wo.astype(BF16), res=x2, interpret=interpret)

    qc = _mm(h1, ca_Wq.astype(BF16), ss=ca_ss, ln=True,
             rows_per_batch=s, interpret=interpret)
    y2 = y.reshape(B * sy, D)
    kv = _mm(y2, jnp.concatenate([ca_Wk, ca_Wv], axis=1).astype(BF16),
             interpret=interpret)
    o2 = _attn(qc, kv, B=B, H=H, dh=dh, sq=s, sk=sy,
               qc0=0, kc0=0, vc0=H // 2, interpret=interpret)
    h2 = _mm(o2, ca_Wo.astype(BF16), res=h1, interpret=interpret)

    gate_pad = jnp.pad(gate_w.T, ((0, 0), (0, P - E)))
    xn, coef = _pre_moe(h2, moe_ss, gate_pad, lt_pad, rows_per_batch=s, E=E,
                        interpret=interpret)

    bm = _fit(1024, S)
    PMAX = E * S
    maskT = (coef[:, :E] > 0.0).T
    cum = jnp.cumsum(maskT.astype(jnp.int32), axis=1)
    g = cum[:, -1]
    nb = (g + bm - 1) // bm
    off_blk = jnp.concatenate(
        [jnp.zeros((1,), jnp.int32), jnp.cumsum(nb)[:-1].astype(jnp.int32)])
    dst = off_blk[:, None] * bm + cum - 1
    dst = jnp.where(maskT, dst, PMAX).reshape(-1)
    tok = jnp.broadcast_to(jnp.arange(S, dtype=jnp.int32), (E, S)).reshape(-1)
    tok_g = jnp.zeros((PMAX,), jnp.int32).at[dst].set(tok, mode="drop")
    coef_g = jnp.zeros((PMAX,), F32).at[dst].set(
        coef[:, :E].T.reshape(-1), mode="drop")
    coef_g8 = jnp.tile(coef_g[:, None], (1, 8))

    xg = _sc_gather(xn.reshape(S, D // 128, 128), tok_g, PMAX,
                    interpret=interpret).reshape(PMAX, D)
    eo_g = _gmoe(xg, exp_W1, exp_b1, exp_W2, exp_b2, coef_g8,
                 nb.astype(jnp.int32), off_blk, bm=bm, interpret=interpret)
    geall = _sc_gather(eo_g.reshape(PMAX + bm, D // 128, 128), dst, PMAX,
                       interpret=interpret).reshape(PMAX, D)
    out = _combine(h2, coef, geall, E, interpret=interpret)
    return out.reshape(B, s, D)


def kernel(x, y, emb, sa_Wqkv, sa_Wo, sa_modW, sa_modb, ca_Wq, ca_Wk, ca_Wv,
           ca_Wo, ca_modW, ca_modb, gate_w, text_gate_w, moe_modW, moe_modb,
           exp_W1, exp_b1, exp_W2, exp_b2):
    return _forward(x, y, emb, sa_Wqkv, sa_Wo, sa_modW, sa_modb, ca_Wq,
                    ca_Wk, ca_Wv, ca_Wo, ca_modW, ca_modb, gate_w,
                    text_gate_w, moe_modW, moe_modb, exp_W1, exp_b1, exp_W2,
                    exp_b2)


# speedup vs baseline: 2.8485x; 2.8485x over previous
import functools

import jax
import jax.numpy as jnp
from jax.experimental import pallas as pl
from jax.experimental.pallas import tpu as pltpu

F32 = jnp.float32
BF16 = jnp.bfloat16


def _fit(b, n):
    b = min(b, n)
    while n % b:
        b -= 1
    return b


def _mm_body(x_ref, w_ref, ss_ref, b_ref, res_ref, o_ref, *, ln, act, D,
             wcast):
    x = x_ref[...].astype(F32)
    if act == "silu":
        x = x * jax.nn.sigmoid(x)
    if ln:
        mu = jnp.mean(x, axis=1, keepdims=True)
        var = jnp.mean((x - mu) * (x - mu), axis=1, keepdims=True)
        xh = (x - mu) * jax.lax.rsqrt(var + 1e-5)
        ss = ss_ref[0]
        x = xh * (1.0 + ss[:, D:]) + ss[:, :D]
    w = w_ref[...]
    if wcast:
        x = x.astype(BF16)
        w = w.astype(BF16)
    out = jnp.dot(x, w, preferred_element_type=F32)
    if b_ref is not None:
        out = out + b_ref[0]
    if res_ref is not None:
        out = out + res_ref[...]
    o_ref[...] = out


def _mm(x, w, ss=None, bias=None, res=None, *, ln=False, act=None,
        bm=512, bn=512, rows_per_batch=None, wcast=False, interpret=False):
    M, K = x.shape
    N = w.shape[1]
    bm = _fit(bm, M)
    if rows_per_batch is not None:
        bm = _fit(bm, rows_per_batch)
    bn = _fit(bn, N)
    grid = (M // bm, N // bn)
    in_specs = [
        pl.BlockSpec((bm, K), lambda m, n: (m, 0)),
        pl.BlockSpec((K, bn), lambda m, n: (0, n)),
    ]
    args = [x, w]
    if ss is not None:
        rpb = rows_per_batch // bm
        in_specs.append(pl.BlockSpec((1, 1, ss.shape[-1]),
                                     lambda m, n: (m // rpb, 0, 0)))
        args.append(ss)
    if bias is not None:
        in_specs.append(pl.BlockSpec((1, 1, bn), lambda m, n: (0, 0, n)))
        args.append(bias.reshape(1, 1, N))
    if res is not None:
        in_specs.append(pl.BlockSpec((bm, bn), lambda m, n: (m, n)))
        args.append(res)

    def body(*refs):
        x_ref, w_ref = refs[0], refs[1]
        i = 2
        ss_ref = b_ref = res_ref = None
        if ss is not None:
            ss_ref = refs[i]; i += 1
        if bias is not None:
            b_ref = refs[i]; i += 1
        if res is not None:
            res_ref = refs[i]; i += 1
        _mm_body(x_ref, w_ref, ss_ref, b_ref, res_ref, refs[-1],
                 ln=ln, act=act, D=K, wcast=wcast)

    return pl.pallas_call(
        body,
        grid=grid,
        in_specs=in_specs,
        out_specs=pl.BlockSpec((bm, bn), lambda m, n: (m, n)),
        out_shape=jax.ShapeDtypeStruct((M, N), F32),
        compiler_params=pltpu.CompilerParams(
            dimension_semantics=("parallel", "parallel")),
        interpret=interpret,
    )(*args)


def _attn_body(q_ref, k_ref, v_ref, o_ref, *, scale, dh):
    qq = (q_ref[...].astype(F32) * scale).astype(BF16)
    kk = k_ref[...].astype(BF16)
    vv = v_ref[...].astype(BF16)
    outs = []
    for i in (0, 1):
        q = qq[:, i * dh:(i + 1) * dh]
        k = kk[:, i * dh:(i + 1) * dh]
        v = vv[:, i * dh:(i + 1) * dh]
        s = jax.lax.dot_general(q, k, (((1,), (1,)), ((), ())),
                                preferred_element_type=F32)
        m = jnp.max(s, axis=1, keepdims=True)
        p = jnp.exp(s - m)
        l = jnp.sum(p, axis=1, keepdims=True)
        p = (p / l).astype(BF16)
        outs.append(jnp.dot(p, v, preferred_element_type=F32))
    o_ref[...] = jnp.concatenate(outs, axis=1)


def _attn(q_arr, kv_arr, *, B, H, dh, sq, sk, qc0, kc0, vc0, bq=512,
          interpret=False):
    bq = _fit(bq, sq)
    nq = sq // bq
    HP = H // 2
    grid = (B * HP, nq)
    return pl.pallas_call(
        functools.partial(_attn_body, scale=1.0 / (dh ** 0.5), dh=dh),
        grid=grid,
        in_specs=[
            pl.BlockSpec((bq, 2 * dh),
                         lambda bh, i: ((bh // HP) * nq + i, qc0 + bh % HP)),
            pl.BlockSpec((sk, 2 * dh),
                         lambda bh, i: (bh // HP, kc0 + bh % HP)),
            pl.BlockSpec((sk, 2 * dh),
                         lambda bh, i: (bh // HP, vc0 + bh % HP)),
        ],
        out_specs=pl.BlockSpec((bq, 2 * dh),
                               lambda bh, i: ((bh // HP) * nq + i, bh % HP)),
        out_shape=jax.ShapeDtypeStruct((B * sq, H * dh), F32),
        compiler_params=pltpu.CompilerParams(
            dimension_semantics=("parallel", "parallel")),
        interpret=interpret,
    )(q_arr, kv_arr, kv_arr)


def _pre_moe_body(h2_ref, ss_ref, gw_ref, lt_ref, xn_ref, coef_ref, *, D, E):
    h2 = h2_ref[...]
    mu = jnp.mean(h2, axis=1, keepdims=True)
    var = jnp.mean((h2 - mu) * (h2 - mu), axis=1, keepdims=True)
    xh = (h2 - mu) * jax.lax.rsqrt(var + 1e-5)
    ss = ss_ref[0]
    xn_ref[...] = (xh * (1.0 + ss[:, D:]) + ss[:, :D]).astype(BF16)

    logits = jnp.dot(h2, gw_ref[...], preferred_element_type=F32) + lt_ref[0]
    col = jax.lax.broadcasted_iota(jnp.int32, logits.shape, 1)
    valid = col < E
    neg = jnp.float32(-1e30)
    lg = jnp.where(valid, logits, neg)
    mx = jnp.max(lg, axis=1, keepdims=True)
    ex = jnp.where(valid, jnp.exp(lg - mx), 0.0)
    scores = ex / jnp.sum(ex, axis=1, keepdims=True)
    gate_on = (jax.nn.sigmoid(logits) - 0.5 > 0.0) & valid
    coef_ref[...] = jnp.where(gate_on, scores, 0.0)


def _pre_moe(h2, moe_ss, gw_pad, lt_pad, *, rows_per_batch, E, bm=512,
             interpret=False):
    S, D = h2.shape
    P = gw_pad.shape[1]
    bm = _fit(bm, rows_per_batch)
    rpb = rows_per_batch // bm
    return pl.pallas_call(
        functools.partial(_pre_moe_body, D=D, E=E),
        grid=(S // bm,),
        in_specs=[
            pl.BlockSpec((bm, D), lambda m: (m, 0)),
            pl.BlockSpec((1, 1, 2 * D), lambda m: (m // rpb, 0, 0)),
            pl.BlockSpec((D, P), lambda m: (0, 0)),
            pl.BlockSpec((1, 1, P), lambda m: (m // rpb, 0, 0)),
        ],
        out_specs=[
            pl.BlockSpec((bm, D), lambda m: (m, 0)),
            pl.BlockSpec((bm, P), lambda m: (m, 0)),
        ],
        out_shape=[
            jax.ShapeDtypeStruct((S, D), BF16),
            jax.ShapeDtypeStruct((S, P), F32),
        ],
        compiler_params=pltpu.CompilerParams(
            dimension_semantics=("parallel",)),
        interpret=interpret,
    )(h2, moe_ss, gw_pad, lt_pad)


def _moe_body(xn_ref, w1_ref, b1_ref, w2_ref, b2_ref, coef_ref, o_ref,
              acc_ref, sem, *, C, E, bm):
    e = pl.program_id(0)
    c = pl.program_id(1)
    mi = pl.program_id(2)
    base = mi * bm

    h = jnp.dot(xn_ref[pl.ds(base, bm), :], w1_ref[0].astype(BF16),
                preferred_element_type=F32) + b1_ref[0]
    h = jax.nn.gelu(h, approximate=True)
    eo = jnp.dot(h.astype(BF16), w2_ref[0].astype(BF16),
                 preferred_element_type=F32)
    is_last_c = (c == C - 1).astype(F32)
    eo = eo + is_last_c * b2_ref[0]
    cb = coef_ref[pl.ds(base, bm), :]
    lane = jax.lax.broadcasted_iota(jnp.int32, cb.shape, 1)
    cf = jnp.sum(jnp.where(lane == e, cb, 0.0), axis=1, keepdims=True)
    contrib = cf * eo

    @pl.when((e == 0) & (c == 0))
    def _init():
        acc_ref[pl.ds(base, bm), :] = contrib

    @pl.when((e > 0) | (c > 0))
    def _accum():
        acc_ref[pl.ds(base, bm), :] += contrib

    @pl.when((e == E - 1) & (c == C - 1))
    def _flush():
        cp = pltpu.make_async_copy(acc_ref.at[pl.ds(base, bm), :],
                                   o_ref.at[pl.ds(base, bm), :], sem)
        cp.start()
        cp.wait()


def _moe(xn, w1, b1, w2, b2, coef, *, bm=1024, bc=1024, interpret=False):
    S, D = xn.shape
    E, _, ff = w1.shape
    P = coef.shape[1]
    bm = _fit(bm, S)
    bc = _fit(bc, ff)
    C = ff // bc
    MI = S // bm
    grid = (E, C, MI)
    return pl.pallas_call(
        functools.partial(_moe_body, C=C, E=E, bm=bm),
        grid=grid,
        in_specs=[
            pl.BlockSpec((S, D), lambda e, c, m: (0, 0)),
            pl.BlockSpec((1, D, bc), lambda e, c, m: (e, 0, c)),
            pl.BlockSpec((1, 1, bc), lambda e, c, m: (e, 0, c)),
            pl.BlockSpec((1, bc, D), lambda e, c, m: (e, c, 0)),
            pl.BlockSpec((1, 1, D), lambda e, c, m: (e, 0, 0)),
            pl.BlockSpec((S, P), lambda e, c, m: (0, 0)),
        ],
        out_specs=pl.BlockSpec(memory_space=pltpu.MemorySpace.HBM),
        out_shape=jax.ShapeDtypeStruct((S, D), F32),
        scratch_shapes=[
            pltpu.VMEM((S, D), F32),
            pltpu.SemaphoreType.DMA,
        ],
        compiler_params=pltpu.CompilerParams(
            dimension_semantics=("arbitrary", "arbitrary", "arbitrary")),
        interpret=interpret,
    )(xn, w1, b1.reshape(E, 1, ff), w2, b2.reshape(E, 1, D), coef)


def _add_body(a_ref, b_ref, o_ref):
    o_ref[...] = a_ref[...] + b_ref[...]


def _add(a, b, *, bm=1024, interpret=False):
    M, N = a.shape
    bm = _fit(bm, M)
    return pl.pallas_call(
        _add_body,
        grid=(M // bm,),
        in_specs=[pl.BlockSpec((bm, N), lambda m: (m, 0)),
                  pl.BlockSpec((bm, N), lambda m: (m, 0))],
        out_specs=pl.BlockSpec((bm, N), lambda m: (m, 0)),
        out_shape=jax.ShapeDtypeStruct((M, N), F32),
        compiler_params=pltpu.CompilerParams(
            dimension_semantics=("parallel",)),
        interpret=interpret,
    )(a, b)


def _forward(x, y, emb, sa_Wqkv, sa_Wo, sa_modW, sa_modb, ca_Wq, ca_Wk,
             ca_Wv, ca_Wo, ca_modW, ca_modb, gate_w, text_gate_w, moe_modW,
             moe_modb, exp_W1, exp_b1, exp_W2, exp_b2, interpret=False):
    H = 16
    E = gate_w.shape[0]
    B, s, D = x.shape
    dh = D // H
    sy = y.shape[1]
    S = B * s
    P = 128

    modW = jnp.concatenate([sa_modW, ca_modW, moe_modW], axis=1)
    modb = jnp.concatenate([sa_modb, ca_modb, moe_modb])
    ss_all = _mm(emb, modW, bias=modb, act="silu", bm=B, bn=1024,
                 interpret=interpret)
    sa_ss = ss_all[:, 0 * D:2 * D].reshape(B, 1, 2 * D)
    ca_ss = ss_all[:, 2 * D:4 * D].reshape(B, 1, 2 * D)
    moe_ss = ss_all[:, 4 * D:6 * D].reshape(B, 1, 2 * D)

    gw_pad = jnp.pad(text_gate_w.T, ((0, 0), (0, P - E)))
    lt_pad = _mm(emb, gw_pad, bm=B, bn=P, interpret=interpret).reshape(B, 1, P)

    x2 = x.reshape(S, D)
    qkv = _mm(x2, sa_Wqkv, ss=sa_ss, ln=True, bn=3 * D, wcast=True,
              rows_per_batch=s, interpret=interpret)
    o = _attn(qkv, qkv, B=B, H=H, dh=dh, sq=s, sk=s,
              qc0=0, kc0=H // 2, vc0=H, interpret=interpret)
    h1 = _mm(o, sa_Wo, res=x2, bn=D, wcast=True, interpret=interpret)

    qc = _mm(h1, ca_Wq, ss=ca_ss, ln=True, bn=D, wcast=True,
             rows_per_batch=s, interpret=interpret)
    y2 = y.reshape(B * sy, D)
    kv = _mm(y2, jnp.concatenate([ca_Wk, ca_Wv], axis=1), bn=2 * D,
             wcast=True, interpret=interpret)
    o2 = _attn(qc, kv, B=B, H=H, dh=dh, sq=s, sk=sy,
               qc0=0, kc0=0, vc0=H // 2, interpret=interpret)
    h2 = _mm(o2, ca_Wo, res=h1, bn=D, wcast=True, interpret=interpret)

    gate_pad = jnp.pad(gate_w.T, ((0, 0), (0, P - E)))
    xn, coef = _pre_moe(h2, moe_ss, gate_pad, lt_pad, rows_per_batch=s, E=E,
                        interpret=interpret)

    ymoe = _moe(xn, exp_W1, exp_b1, exp_W2, exp_b2, coef,
                interpret=interpret)
    out = _add(ymoe, h2, interpret=interpret)
    return out.reshape(B, s, D)


def kernel(x, y, emb, sa_Wqkv, sa_Wo, sa_modW, sa_modb, ca_Wq, ca_Wk, ca_Wv,
           ca_Wo, ca_modW, ca_modb, gate_w, text_gate_w, moe_modW, moe_modb,
           exp_W1, exp_b1, exp_W2, exp_b2):
    return _forward(x, y, emb, sa_Wqkv, sa_Wo, sa_modW, sa_modb, ca_Wq,
                    ca_Wk, ca_Wv, ca_Wo, ca_modW, ca_modb, gate_w,
                    text_gate_w, moe_modW, moe_modb, exp_W1, exp_b1, exp_W2,
                    exp_b2)


# speedup vs baseline: 2.8752x; 1.0094x over previous
import functools

import jax
import jax.numpy as jnp
from jax.experimental import pallas as pl
from jax.experimental.pallas import tpu as pltpu

F32 = jnp.float32
BF16 = jnp.bfloat16


def _fit(b, n):
    b = min(b, n)
    while n % b:
        b -= 1
    return b


def _mm_body(x_ref, w_ref, ss_ref, b_ref, res_ref, o_ref, *, ln, act, D,
             wcast):
    x = x_ref[...].astype(F32)
    if act == "silu":
        x = x * jax.nn.sigmoid(x)
    if ln:
        mu = jnp.mean(x, axis=1, keepdims=True)
        var = jnp.mean((x - mu) * (x - mu), axis=1, keepdims=True)
        xh = (x - mu) * jax.lax.rsqrt(var + 1e-5)
        ss = ss_ref[0]
        x = xh * (1.0 + ss[:, D:]) + ss[:, :D]
    w = w_ref[...]
    if wcast:
        x = x.astype(BF16)
        w = w.astype(BF16)
    out = jnp.dot(x, w, preferred_element_type=F32)
    if b_ref is not None:
        out = out + b_ref[0]
    if res_ref is not None:
        out = out + res_ref[...]
    o_ref[...] = out


def _mm(x, w, ss=None, bias=None, res=None, *, ln=False, act=None,
        bm=512, bn=512, rows_per_batch=None, wcast=False, interpret=False):
    M, K = x.shape
    N = w.shape[1]
    bm = _fit(bm, M)
    if rows_per_batch is not None:
        bm = _fit(bm, rows_per_batch)
    bn = _fit(bn, N)
    grid = (M // bm, N // bn)
    in_specs = [
        pl.BlockSpec((bm, K), lambda m, n: (m, 0)),
        pl.BlockSpec((K, bn), lambda m, n: (0, n)),
    ]
    args = [x, w]
    if ss is not None:
        rpb = rows_per_batch // bm
        in_specs.append(pl.BlockSpec((1, 1, ss.shape[-1]),
                                     lambda m, n: (m // rpb, 0, 0)))
        args.append(ss)
    if bias is not None:
        in_specs.append(pl.BlockSpec((1, 1, bn), lambda m, n: (0, 0, n)))
        args.append(bias.reshape(1, 1, N))
    if res is not None:
        in_specs.append(pl.BlockSpec((bm, bn), lambda m, n: (m, n)))
        args.append(res)

    def body(*refs):
        x_ref, w_ref = refs[0], refs[1]
        i = 2
        ss_ref = b_ref = res_ref = None
        if ss is not None:
            ss_ref = refs[i]; i += 1
        if bias is not None:
            b_ref = refs[i]; i += 1
        if res is not None:
            res_ref = refs[i]; i += 1
        _mm_body(x_ref, w_ref, ss_ref, b_ref, res_ref, refs[-1],
                 ln=ln, act=act, D=K, wcast=wcast)

    return pl.pallas_call(
        body,
        grid=grid,
        in_specs=in_specs,
        out_specs=pl.BlockSpec((bm, bn), lambda m, n: (m, n)),
        out_shape=jax.ShapeDtypeStruct((M, N), F32),
        compiler_params=pltpu.CompilerParams(
            dimension_semantics=("parallel", "parallel")),
        interpret=interpret,
    )(*args)


def _attn_body(q_ref, k_ref, v_ref, o_ref, *, scale, dh):
    qq = (q_ref[...].astype(F32) * scale).astype(BF16)
    kk = k_ref[...].astype(BF16)
    vv = v_ref[...].astype(BF16)
    outs = []
    for i in (0, 1):
        q = qq[:, i * dh:(i + 1) * dh]
        k = kk[:, i * dh:(i + 1) * dh]
        v = vv[:, i * dh:(i + 1) * dh]
        s = jax.lax.dot_general(q, k, (((1,), (1,)), ((), ())),
                                preferred_element_type=F32)
        m = jnp.max(s, axis=1, keepdims=True)
        p = jnp.exp(s - m)
        l = jnp.sum(p, axis=1, keepdims=True)
        p = (p / l).astype(BF16)
        outs.append(jnp.dot(p, v, preferred_element_type=F32))
    o_ref[...] = jnp.concatenate(outs, axis=1)


def _attn(q_arr, kv_arr, *, B, H, dh, sq, sk, qc0, kc0, vc0, bq=1024,
          interpret=False):
    bq = _fit(bq, sq)
    nq = sq // bq
    HP = H // 2
    grid = (B * HP, nq)
    return pl.pallas_call(
        functools.partial(_attn_body, scale=1.0 / (dh ** 0.5), dh=dh),
        grid=grid,
        in_specs=[
            pl.BlockSpec((bq, 2 * dh),
                         lambda bh, i: ((bh // HP) * nq + i, qc0 + bh % HP)),
            pl.BlockSpec((sk, 2 * dh),
                         lambda bh, i: (bh // HP, kc0 + bh % HP)),
            pl.BlockSpec((sk, 2 * dh),
                         lambda bh, i: (bh // HP, vc0 + bh % HP)),
        ],
        out_specs=pl.BlockSpec((bq, 2 * dh),
                               lambda bh, i: ((bh // HP) * nq + i, bh % HP)),
        out_shape=jax.ShapeDtypeStruct((B * sq, H * dh), F32),
        compiler_params=pltpu.CompilerParams(
            dimension_semantics=("parallel", "parallel")),
        interpret=interpret,
    )(q_arr, kv_arr, kv_arr)


def _pre_moe_body(h2_ref, ss_ref, gw_ref, lt_ref, xn_ref, coef_ref, *, D, E):
    h2 = h2_ref[...]
    mu = jnp.mean(h2, axis=1, keepdims=True)
    var = jnp.mean((h2 - mu) * (h2 - mu), axis=1, keepdims=True)
    xh = (h2 - mu) * jax.lax.rsqrt(var + 1e-5)
    ss = ss_ref[0]
    xn_ref[...] = (xh * (1.0 + ss[:, D:]) + ss[:, :D]).astype(BF16)

    logits = jnp.dot(h2, gw_ref[...], preferred_element_type=F32) + lt_ref[0]
    col = jax.lax.broadcasted_iota(jnp.int32, logits.shape, 1)
    valid = col < E
    neg = jnp.float32(-1e30)
    lg = jnp.where(valid, logits, neg)
    mx = jnp.max(lg, axis=1, keepdims=True)
    ex = jnp.where(valid, jnp.exp(lg - mx), 0.0)
    scores = ex / jnp.sum(ex, axis=1, keepdims=True)
    gate_on = (jax.nn.sigmoid(logits) - 0.5 > 0.0) & valid
    coef_ref[...] = jnp.where(gate_on, scores, 0.0)


def _pre_moe(h2, moe_ss, gw_pad, lt_pad, *, rows_per_batch, E, bm=512,
             interpret=False):
    S, D = h2.shape
    P = gw_pad.shape[1]
    bm = _fit(bm, rows_per_batch)
    rpb = rows_per_batch // bm
    return pl.pallas_call(
        functools.partial(_pre_moe_body, D=D, E=E),
        grid=(S // bm,),
        in_specs=[
            pl.BlockSpec((bm, D), lambda m: (m, 0)),
            pl.BlockSpec((1, 1, 2 * D), lambda m: (m // rpb, 0, 0)),
            pl.BlockSpec((D, P), lambda m: (0, 0)),
            pl.BlockSpec((1, 1, P), lambda m: (m // rpb, 0, 0)),
        ],
        out_specs=[
            pl.BlockSpec((bm, D), lambda m: (m, 0)),
            pl.BlockSpec((bm, P), lambda m: (m, 0)),
        ],
        out_shape=[
            jax.ShapeDtypeStruct((S, D), BF16),
            jax.ShapeDtypeStruct((S, P), F32),
        ],
        compiler_params=pltpu.CompilerParams(
            dimension_semantics=("parallel",)),
        interpret=interpret,
    )(h2, moe_ss, gw_pad, lt_pad)


def _moe_body(xn_ref, w1_ref, b1_ref, w2_ref, b2_ref, coef_ref, o_ref,
              acc_ref, sem, *, C, E, bm):
    e = pl.program_id(0)
    c = pl.program_id(1)
    mi = pl.program_id(2)
    base = mi * bm

    h = jnp.dot(xn_ref[pl.ds(base, bm), :], w1_ref[0].astype(BF16),
                preferred_element_type=F32) + b1_ref[0]
    h = jax.nn.gelu(h, approximate=True)
    eo = jnp.dot(h.astype(BF16), w2_ref[0].astype(BF16),
                 preferred_element_type=F32)
    is_last_c = (c == C - 1).astype(F32)
    eo = eo + is_last_c * b2_ref[0]
    cb = coef_ref[pl.ds(base, bm), :]
    lane = jax.lax.broadcasted_iota(jnp.int32, cb.shape, 1)
    cf = jnp.sum(jnp.where(lane == e, cb, 0.0), axis=1, keepdims=True)
    contrib = cf * eo

    @pl.when((e == 0) & (c == 0))
    def _init():
        acc_ref[pl.ds(base, bm), :] = contrib

    @pl.when((e > 0) | (c > 0))
    def _accum():
        acc_ref[pl.ds(base, bm), :] += contrib

    @pl.when((e == E - 1) & (c == C - 1))
    def _flush():
        cp = pltpu.make_async_copy(acc_ref.at[pl.ds(base, bm), :],
                                   o_ref.at[pl.ds(base, bm), :], sem)
        cp.start()
        cp.wait()


def _moe(xn, w1, b1, w2, b2, coef, *, bm=2048, bc=1024, interpret=False):
    S, D = xn.shape
    E, _, ff = w1.shape
    P = coef.shape[1]
    bm = _fit(bm, S)
    bc = _fit(bc, ff)
    C = ff // bc
    MI = S // bm
    grid = (E, C, MI)
    return pl.pallas_call(
        functools.partial(_moe_body, C=C, E=E, bm=bm),
        grid=grid,
        in_specs=[
            pl.BlockSpec((S, D), lambda e, c, m: (0, 0)),
            pl.BlockSpec((1, D, bc), lambda e, c, m: (e, 0, c)),
            pl.BlockSpec((1, 1, bc), lambda e, c, m: (e, 0, c)),
            pl.BlockSpec((1, bc, D), lambda e, c, m: (e, c, 0)),
            pl.BlockSpec((1, 1, D), lambda e, c, m: (e, 0, 0)),
            pl.BlockSpec((S, P), lambda e, c, m: (0, 0)),
        ],
        out_specs=pl.BlockSpec(memory_space=pltpu.MemorySpace.HBM),
        out_shape=jax.ShapeDtypeStruct((S, D), F32),
        scratch_shapes=[
            pltpu.VMEM((S, D), F32),
            pltpu.SemaphoreType.DMA,
        ],
        compiler_params=pltpu.CompilerParams(
            dimension_semantics=("arbitrary", "arbitrary", "arbitrary")),
        interpret=interpret,
    )(xn, w1, b1.reshape(E, 1, ff), w2, b2.reshape(E, 1, D), coef)


def _add_body(a_ref, b_ref, o_ref):
    o_ref[...] = a_ref[...] + b_ref[...]


def _add(a, b, *, bm=1024, interpret=False):
    M, N = a.shape
    bm = _fit(bm, M)
    return pl.pallas_call(
        _add_body,
        grid=(M // bm,),
        in_specs=[pl.BlockSpec((bm, N), lambda m: (m, 0)),
                  pl.BlockSpec((bm, N), lambda m: (m, 0))],
        out_specs=pl.BlockSpec((bm, N), lambda m: (m, 0)),
        out_shape=jax.ShapeDtypeStruct((M, N), F32),
        compiler_params=pltpu.CompilerParams(
            dimension_semantics=("parallel",)),
        interpret=interpret,
    )(a, b)


def _forward(x, y, emb, sa_Wqkv, sa_Wo, sa_modW, sa_modb, ca_Wq, ca_Wk,
             ca_Wv, ca_Wo, ca_modW, ca_modb, gate_w, text_gate_w, moe_modW,
             moe_modb, exp_W1, exp_b1, exp_W2, exp_b2, interpret=False):
    H = 16
    E = gate_w.shape[0]
    B, s, D = x.shape
    dh = D // H
    sy = y.shape[1]
    S = B * s
    P = 128

    modW = jnp.concatenate([sa_modW, ca_modW, moe_modW], axis=1)
    modb = jnp.concatenate([sa_modb, ca_modb, moe_modb])
    ss_all = _mm(emb, modW, bias=modb, act="silu", bm=B, bn=1024,
                 interpret=interpret)
    sa_ss = ss_all[:, 0 * D:2 * D].reshape(B, 1, 2 * D)
    ca_ss = ss_all[:, 2 * D:4 * D].reshape(B, 1, 2 * D)
    moe_ss = ss_all[:, 4 * D:6 * D].reshape(B, 1, 2 * D)

    gw_pad = jnp.pad(text_gate_w.T, ((0, 0), (0, P - E)))
    lt_pad = _mm(emb, gw_pad, bm=B, bn=P, interpret=interpret).reshape(B, 1, P)

    x2 = x.reshape(S, D)
    qkv = _mm(x2, sa_Wqkv, ss=sa_ss, ln=True, bn=3 * D, wcast=True,
              rows_per_batch=s, interpret=interpret)
    o = _attn(qkv, qkv, B=B, H=H, dh=dh, sq=s, sk=s,
              qc0=0, kc0=H // 2, vc0=H, interpret=interpret)
    h1 = _mm(o, sa_Wo, res=x2, bn=D, wcast=True, interpret=interpret)

    qc = _mm(h1, ca_Wq, ss=ca_ss, ln=True, bn=D, wcast=True,
             rows_per_batch=s, interpret=interpret)
    y2 = y.reshape(B * sy, D)
    kv = _mm(y2, jnp.concatenate([ca_Wk, ca_Wv], axis=1), bn=2 * D,
             wcast=True, interpret=interpret)
    o2 = _attn(qc, kv, B=B, H=H, dh=dh, sq=s, sk=sy,
               qc0=0, kc0=0, vc0=H // 2, interpret=interpret)
    h2 = _mm(o2, ca_Wo, res=h1, bn=D, wcast=True, interpret=interpret)

    gate_pad = jnp.pad(gate_w.T, ((0, 0), (0, P - E)))
    xn, coef = _pre_moe(h2, moe_ss, gate_pad, lt_pad, rows_per_batch=s, E=E,
                        interpret=interpret)

    ymoe = _moe(xn, exp_W1, exp_b1, exp_W2, exp_b2, coef,
                interpret=interpret)
    out = _add(ymoe, h2, interpret=interpret)
    return out.reshape(B, s, D)


def kernel(x, y, emb, sa_Wqkv, sa_Wo, sa_modW, sa_modb, ca_Wq, ca_Wk, ca_Wv,
           ca_Wo, ca_modW, ca_modb, gate_w, text_gate_w, moe_modW, moe_modb,
           exp_W1, exp_b1, exp_W2, exp_b2):
    return _forward(x, y, emb, sa_Wqkv, sa_Wo, sa_modW, sa_modb, ca_Wq,
                    ca_Wk, ca_Wv, ca_Wo, ca_modW, ca_modb, gate_w,
                    text_gate_w, moe_modW, moe_modb, exp_W1, exp_b1, exp_W2,
                    exp_b2)
